# Initial kernel scaffold; baseline (speedup 1.0000x reference)
import jax
import jax.numpy as jnp
from jax.experimental import pallas as pl


def kernel(x, conv_layer_w, conv_layer_b, block0_w, block0_b, block1_w, block1_b, block2_w, block2_b, block3_w, block3_b, block4_w, block4_b, block5_w, block5_b, block6_w, block6_b, upscaler_w, upscaler_b):
    raise NotImplementedError("write your pallas kernel here")


# speedup vs baseline: 1.3920x; 1.3920x over previous
import jax
import jax.numpy as jnp
from jax.experimental import pallas as pl
from jax.experimental.pallas import tpu as pltpu

_DEPTH = 7
_MID = 4
_C8 = 2 * _MID
_KB = 9 * 2 * _C8
_KU = _DEPTH * _KB
_SLOT = 16
_UP_ROW = _SLOT * (1 + _DEPTH)
_SLAB_ROWS = _UP_ROW + _SLOT
_SLAB_COLS = 1024


def _cnn_kernel_body(h, w, n2):
    taps = [(t // 3 - 1, t % 3 - 1) for t in range(9)]

    def body(x_ref, res_ref, w_ref, b_ref, m_ref, out_ref,
             xt_ref, ct_ref, ft_ref):

        def write_taps(act_bf, dst_ref, base):
            for t, (di, dj) in enumerate(taps):
                s = di * w + dj
                if s == 0:
                    dst_ref[base + 16 * t: base + 16 * (t + 1), :] = act_bf
                else:
                    v = pltpu.roll(act_bf, (-s) % n2, axis=1)
                    dst_ref[base + 16 * t: base + 16 * (t + 1), :] = (
                        v * m_ref[16 * t:16 * (t + 1), :])

        x_bf = x_ref[0].astype(jnp.bfloat16)
        for t, (di, dj) in enumerate(taps):
            s = di * w + dj
            if s == 0:
                xt_ref[2 * t:2 * t + 2, :] = x_bf
            else:
                v = pltpu.roll(x_bf, (-s) % n2, axis=1)
                xt_ref[2 * t:2 * t + 2, :] = v * m_ref[16 * t:16 * t + 2, :]

        h0 = jnp.dot(w_ref[0:_SLOT, 0:18], xt_ref[0:18, :],
                     preferred_element_type=jnp.float32)
        h0 = h0 + b_ref[0:_SLOT, :]
        write_taps(h0.astype(jnp.bfloat16), ct_ref, 0)

        for i in range(_DEPTH):
            rhs = ct_ref[...] if i == 0 else ft_ref[_KB * (i - 1):_KB * i, :]
            r = _SLOT * (1 + i)
            y = jnp.dot(w_ref[r:r + _SLOT, 0:_KB], rhs,
                        preferred_element_type=jnp.float32)
            crelu = jnp.maximum(y + b_ref[r:r + _SLOT, :], 0.0)
            write_taps(crelu.astype(jnp.bfloat16), ft_ref, _KB * i)

        up = jnp.dot(w_ref[_UP_ROW:_UP_ROW + 8, 0:_KU], ft_ref[...],
                     preferred_element_type=jnp.float32)
        up = up + b_ref[_UP_ROW:_UP_ROW + 8, :]
        out_ref[0] = (up + res_ref[0]).astype(out_ref.dtype)

    return body


def _fold_upscaler(w_up):
    m = jnp.array([[[1., 0., 0.], [0., 1., 1.], [0., 0., 0.]],
                   [[0., 0., 0.], [1., 1., 0.], [0., 0., 1.]]],
                  dtype=w_up.dtype)
    k = jnp.einsum("pia,qjb,ocab->pqocij", m, m, w_up)
    k = jnp.transpose(k[:, :, 0], (3, 4, 0, 1, 2))
    return k.reshape(9, 4, w_up.shape[1])


def _pack_slabs(params):
    wslab = jnp.zeros((_SLAB_ROWS, _SLAB_COLS), jnp.float32)
    bslab = jnp.zeros((_SLAB_ROWS, 1), jnp.float32)

    w0 = params["conv_layer"]["w"].reshape(_C8, 9)
    s0 = jnp.zeros((_SLOT, 9, 2), jnp.float32)
    for p in range(2):
        s0 = s0.at[_C8 * p:_C8 * (p + 1), :, p].set(w0)
    wslab = wslab.at[0:_SLOT, 0:18].set(s0.reshape(_SLOT, 18))
    bslab = bslab.at[0:_SLOT, 0].set(
        jnp.tile(params["conv_layer"]["b"], 2))

    for i, blk in enumerate(params["blocks"]):
        wb = jnp.transpose(blk["w"], (0, 2, 3, 1)).reshape(_MID, 9, _C8)
        sb = jnp.zeros((_SLOT, 9, 2, _C8), jnp.float32)
        for p in range(2):
            sb = sb.at[_C8 * p:_C8 * p + _MID, :, p, :].set(wb)
            sb = sb.at[_C8 * p + _MID:_C8 * (p + 1), :, p, :].set(-wb)
        r = _SLOT * (1 + i)
        wslab = wslab.at[r:r + _SLOT, 0:_KB].set(sb.reshape(_SLOT, _KB))
        bb = jnp.concatenate([blk["b"], -blk["b"]])
        bslab = bslab.at[r:r + _SLOT, 0].set(jnp.tile(bb, 2))

    wu = _fold_upscaler(params["upscaler"]["w"])
    wu = jnp.transpose(wu.reshape(9, 4, _DEPTH, _C8), (1, 2, 0, 3))
    su = jnp.zeros((8, _DEPTH, 9, 2, _C8), jnp.float32)
    for p in range(2):
        su = su.at[4 * p:4 * (p + 1), :, :, p, :].set(wu)
    wslab = wslab.at[_UP_ROW:_UP_ROW + 8, 0:_KU].set(su.reshape(8, _KU))
    bslab = bslab.at[_UP_ROW:_UP_ROW + 8, 0].set(params["upscaler"]["b"][0])
    return wslab.astype(jnp.bfloat16), bslab


def _mask_slab(h, w, imgs):
    hw = h * w
    q = jnp.arange(hw, dtype=jnp.int32)
    yy, xx = q // w, q % w
    rows = []
    for t in range(9):
        di, dj = t // 3 - 1, t % 3 - 1
        ok = (yy + di >= 0) & (yy + di < h) & (xx + dj >= 0) & (xx + dj < w)
        rows.append(ok)
    m = jnp.stack(rows).astype(jnp.bfloat16)
    m = jnp.repeat(m, _SLOT, axis=0)
    return jnp.tile(m, (1, imgs))


def _bilinear_x2(x):
    n, c, h, w = x.shape

    def coords(size):
        d = jnp.arange(2 * size, dtype=jnp.float32)
        src = jnp.maximum(0.5 * (d + 0.5) - 0.5, 0.0)
        i0 = jnp.floor(src).astype(jnp.int32)
        i1 = jnp.minimum(i0 + 1, size - 1)
        return i0, i1, src - i0.astype(jnp.float32)

    h0, h1, fh = coords(h)
    w0, w1, fw = coords(w)
    top = (x[:, :, h0, :] * (1.0 - fh)[None, None, :, None]
           + x[:, :, h1, :] * fh[None, None, :, None])
    return (top[:, :, :, w0] * (1.0 - fw)[None, None, None, :]
            + top[:, :, :, w1] * fw[None, None, None, :])


def _pick_bsub(n):
    for b in (8, 4, 2):
        if n % b == 0 and n // b >= 2:
            return b
    return 1


def kernel(x, conv_layer_w, conv_layer_b, block0_w, block0_b, block1_w,
           block1_b, block2_w, block2_b, block3_w, block3_b, block4_w,
           block4_b, block5_w, block5_b, block6_w, block6_b, upscaler_w,
           upscaler_b):
    params = {
        "conv_layer": {"w": conv_layer_w, "b": conv_layer_b},
        "blocks": [
            {"w": block0_w, "b": block0_b}, {"w": block1_w, "b": block1_b},
            {"w": block2_w, "b": block2_b}, {"w": block3_w, "b": block3_b},
            {"w": block4_w, "b": block4_b}, {"w": block5_w, "b": block5_b},
            {"w": block6_w, "b": block6_b},
        ],
        "upscaler": {"w": upscaler_w, "b": upscaler_b},
    }
    n, c, h, w = x.shape
    hw = h * w
    bsub = _pick_bsub(n)
    if bsub == 1:
        raise ValueError("batch must be even")
    g = n // bsub
    half = bsub // 2
    n2 = half * hw

    wslab, bslab = _pack_slabs(params)
    masks = _mask_slab(h, w, half)

    bil = _bilinear_x2(x)
    res = bil.reshape(n, 1, h, 2, w, 2)
    res = jnp.transpose(res, (0, 1, 3, 5, 2, 4)).reshape(n, 4, hw)
    res_g = res.reshape(g, 2, half, 4, hw).transpose(0, 1, 3, 2, 4)
    res_g = res_g.reshape(g, 8, n2)
    x_g = x.reshape(g, 2, n2)

    flops = 2 * n * hw * (_C8 * 9 + _DEPTH * _C8 * _KB // 2 + 2 * _KU)
    ins = (x_g, res_g, wslab, bslab, masks)
    nbytes = sum(a.size * a.dtype.itemsize for a in ins) + n * 4 * hw * 4

    out_par = pl.pallas_call(
        _cnn_kernel_body(h, w, n2),
        out_shape=jax.ShapeDtypeStruct((g, 8, n2), x.dtype),
        grid=(g,),
        in_specs=[
            pl.BlockSpec((1, 2, n2), lambda i: (i, 0, 0)),
            pl.BlockSpec((1, 8, n2), lambda i: (i, 0, 0)),
            pl.BlockSpec((_SLAB_ROWS, _SLAB_COLS), lambda i: (0, 0)),
            pl.BlockSpec((_SLAB_ROWS, 1), lambda i: (0, 0)),
            pl.BlockSpec((_SLAB_ROWS, n2), lambda i: (0, 0)),
        ],
        out_specs=pl.BlockSpec((1, 8, n2), lambda i: (i, 0, 0)),
        scratch_shapes=[
            pltpu.VMEM((32, n2), jnp.bfloat16),
            pltpu.VMEM((_KB, n2), jnp.bfloat16),
            pltpu.VMEM((_KU, n2), jnp.bfloat16),
        ],
        compiler_params=pltpu.CompilerParams(
            dimension_semantics=("parallel",)),
        cost_estimate=pl.CostEstimate(flops=flops, transcendentals=0,
                                      bytes_accessed=nbytes),
    )(*ins)

    out = out_par.reshape(g, 2, 4, half, hw).transpose(0, 1, 3, 2, 4)
    out = out.reshape(n, 4, hw)
    out = out.reshape(n, 2, 2, h, w).transpose(0, 3, 1, 4, 2)
    return out.reshape(n, 1, 2 * h, 2 * w)


# speedup vs baseline: 1.7967x; 1.2907x over previous
import jax
import jax.numpy as jnp
from jax.experimental import pallas as pl
from jax.experimental.pallas import tpu as pltpu

_DEPTH = 7
_MID = 4
_C8 = 2 * _MID
_KB = 9 * 2 * _C8
_KU = _DEPTH * _KB
_SLOT = 16
_UP_ROW = _SLOT * (1 + _DEPTH)
_BIL_ROW = _UP_ROW + _SLOT
_SLAB_ROWS = _BIL_ROW + _SLOT
_SLAB_COLS = 1024
_XTC = 32


def _cnn_kernel_body(h, w, n2):
    taps = [(t // 3 - 1, t % 3 - 1) for t in range(9)]

    def body(x_ref, w_ref, b_ref, m_ref, out_ref,
             xt_ref, ct_ref, ft_ref):

        def write_taps(act_bf, dst_ref, base):
            for t, (di, dj) in enumerate(taps):
                s = di * w + dj
                if s == 0:
                    dst_ref[base + 16 * t: base + 16 * (t + 1), :] = act_bf
                else:
                    v = pltpu.roll(act_bf, (-s) % n2, axis=1)
                    dst_ref[base + 16 * t: base + 16 * (t + 1), :] = (
                        v * m_ref[16 * t:16 * (t + 1), :])

        x_bf = x_ref[0].astype(jnp.bfloat16)
        for t, (di, dj) in enumerate(taps):
            s = di * w + dj
            if s == 0:
                xt_ref[2 * t:2 * t + 2, :] = x_bf
            else:
                v = pltpu.roll(x_bf, (-s) % n2, axis=1)
                xt_ref[2 * t:2 * t + 2, :] = v * m_ref[16 * t:16 * t + 2, :]

        xcol = {}
        for dj in (-1, 0, 1):
            if dj == 0:
                xcol[0] = x_bf
            else:
                r = pltpu.roll(x_bf, (-dj) % n2, axis=1)
                mc = m_ref[16 * (4 + dj):16 * (4 + dj) + 2, :]
                xcol[dj] = x_bf + mc * (r - x_bf)
        for t, (di, dj) in enumerate(taps):
            v = xcol[dj]
            if di != 0:
                r = pltpu.roll(v, (-di * w) % n2, axis=1)
                mr = m_ref[16 * (3 * di + 4):16 * (3 * di + 4) + 2, :]
                v = v + mr * (r - v)
            xt_ref[_XTC + 2 * t:_XTC + 2 * t + 2, :] = v

        h0 = jnp.dot(w_ref[0:_SLOT, 0:18], xt_ref[0:18, :],
                     preferred_element_type=jnp.float32)
        h0 = h0 + b_ref[0:_SLOT, :]
        write_taps(h0.astype(jnp.bfloat16), ct_ref, 0)

        for i in range(_DEPTH):
            rhs = ct_ref[...] if i == 0 else ft_ref[_KB * (i - 1):_KB * i, :]
            r = _SLOT * (1 + i)
            y = jnp.dot(w_ref[r:r + _SLOT, 0:_KB], rhs,
                        preferred_element_type=jnp.float32)
            crelu = jnp.maximum(y + b_ref[r:r + _SLOT, :], 0.0)
            write_taps(crelu.astype(jnp.bfloat16), ft_ref, _KB * i)

        up = jnp.dot(w_ref[_UP_ROW:_UP_ROW + 8, 0:_KU], ft_ref[...],
                     preferred_element_type=jnp.float32)
        up = up + b_ref[_UP_ROW:_UP_ROW + 8, :]

        res = jnp.dot(w_ref[_BIL_ROW:_BIL_ROW + 8, 0:18],
                      xt_ref[_XTC:_XTC + 18, :],
                      preferred_element_type=jnp.float32)
        out_ref[0] = (up + res).astype(out_ref.dtype)

    return body


def _fold_upscaler(w_up):
    m = jnp.array([[[1., 0., 0.], [0., 1., 1.], [0., 0., 0.]],
                   [[0., 0., 0.], [1., 1., 0.], [0., 0., 1.]]],
                  dtype=w_up.dtype)
    k = jnp.einsum("pia,qjb,ocab->pqocij", m, m, w_up)
    k = jnp.transpose(k[:, :, 0], (3, 4, 0, 1, 2))
    return k.reshape(9, 4, w_up.shape[1])


def _pack_slabs(params):
    wslab = jnp.zeros((_SLAB_ROWS, _SLAB_COLS), jnp.float32)
    bslab = jnp.zeros((_SLAB_ROWS, 1), jnp.float32)

    w0 = params["conv_layer"]["w"].reshape(_C8, 9)
    s0 = jnp.zeros((_SLOT, 9, 2), jnp.float32)
    for p in range(2):
        s0 = s0.at[_C8 * p:_C8 * (p + 1), :, p].set(w0)
    wslab = wslab.at[0:_SLOT, 0:18].set(s0.reshape(_SLOT, 18))
    bslab = bslab.at[0:_SLOT, 0].set(
        jnp.tile(params["conv_layer"]["b"], 2))

    for i, blk in enumerate(params["blocks"]):
        wb = jnp.transpose(blk["w"], (0, 2, 3, 1)).reshape(_MID, 9, _C8)
        sb = jnp.zeros((_SLOT, 9, 2, _C8), jnp.float32)
        for p in range(2):
            sb = sb.at[_C8 * p:_C8 * p + _MID, :, p, :].set(wb)
            sb = sb.at[_C8 * p + _MID:_C8 * (p + 1), :, p, :].set(-wb)
        r = _SLOT * (1 + i)
        wslab = wslab.at[r:r + _SLOT, 0:_KB].set(sb.reshape(_SLOT, _KB))
        bb = jnp.concatenate([blk["b"], -blk["b"]])
        bslab = bslab.at[r:r + _SLOT, 0].set(jnp.tile(bb, 2))

    wu = _fold_upscaler(params["upscaler"]["w"])
    wu = jnp.transpose(wu.reshape(9, 4, _DEPTH, _C8), (1, 2, 0, 3))
    su = jnp.zeros((8, _DEPTH, 9, 2, _C8), jnp.float32)
    for p in range(2):
        su = su.at[4 * p:4 * (p + 1), :, :, p, :].set(wu)
    wslab = wslab.at[_UP_ROW:_UP_ROW + 8, 0:_KU].set(su.reshape(8, _KU))
    bslab = bslab.at[_UP_ROW:_UP_ROW + 8, 0].set(params["upscaler"]["b"][0])

    wr = jnp.array([[0.25, 0.75, 0.0], [0.0, 0.75, 0.25]], jnp.float32)
    wb = jnp.einsum("pi,qj->pqij", wr, wr).reshape(4, 9)
    sr = jnp.zeros((8, 9, 2), jnp.float32)
    for p in range(2):
        sr = sr.at[4 * p:4 * (p + 1), :, p].set(wb)
    wslab = wslab.at[_BIL_ROW:_BIL_ROW + 8, 0:18].set(sr.reshape(8, 18))
    return wslab.astype(jnp.bfloat16), bslab


def _mask_slab(h, w, imgs):
    hw = h * w
    q = jnp.arange(hw, dtype=jnp.int32)
    yy, xx = q // w, q % w
    rows = []
    for t in range(9):
        di, dj = t // 3 - 1, t % 3 - 1
        ok = (yy + di >= 0) & (yy + di < h) & (xx + dj >= 0) & (xx + dj < w)
        rows.append(ok)
    m = jnp.stack(rows).astype(jnp.bfloat16)
    m = jnp.repeat(m, _SLOT, axis=0)
    return jnp.tile(m, (1, imgs))


def _pick_bsub(n):
    for b in (8, 4, 2):
        if n % b == 0 and n // b >= 2:
            return b
    return 1


def kernel(x, conv_layer_w, conv_layer_b, block0_w, block0_b, block1_w,
           block1_b, block2_w, block2_b, block3_w, block3_b, block4_w,
           block4_b, block5_w, block5_b, block6_w, block6_b, upscaler_w,
           upscaler_b):
    params = {
        "conv_layer": {"w": conv_layer_w, "b": conv_layer_b},
        "blocks": [
            {"w": block0_w, "b": block0_b}, {"w": block1_w, "b": block1_b},
            {"w": block2_w, "b": block2_b}, {"w": block3_w, "b": block3_b},
            {"w": block4_w, "b": block4_b}, {"w": block5_w, "b": block5_b},
            {"w": block6_w, "b": block6_b},
        ],
        "upscaler": {"w": upscaler_w, "b": upscaler_b},
    }
    n, c, h, w = x.shape
    hw = h * w
    bsub = _pick_bsub(n)
    if bsub == 1:
        raise ValueError("batch must be even")
    g = n // bsub
    half = bsub // 2
    n2 = half * hw

    wslab, bslab = _pack_slabs(params)
    masks = _mask_slab(h, w, half)
    x_g = x.reshape(g, 2, n2)

    flops = 2 * n * hw * (_C8 * 9 + _DEPTH * _C8 * _KB // 2 + 2 * _KU)
    ins = (x_g, wslab, bslab, masks)
    nbytes = sum(a.size * a.dtype.itemsize for a in ins) + n * 4 * hw * 4

    out_par = pl.pallas_call(
        _cnn_kernel_body(h, w, n2),
        out_shape=jax.ShapeDtypeStruct((g, 8, n2), x.dtype),
        grid=(g,),
        in_specs=[
            pl.BlockSpec((1, 2, n2), lambda i: (i, 0, 0)),
            pl.BlockSpec((_SLAB_ROWS, _SLAB_COLS), lambda i: (0, 0)),
            pl.BlockSpec((_SLAB_ROWS, 1), lambda i: (0, 0)),
            pl.BlockSpec((_SLOT * 9, n2), lambda i: (0, 0)),
        ],
        out_specs=pl.BlockSpec((1, 8, n2), lambda i: (i, 0, 0)),
        scratch_shapes=[
            pltpu.VMEM((64, n2), jnp.bfloat16),
            pltpu.VMEM((_KB, n2), jnp.bfloat16),
            pltpu.VMEM((_KU, n2), jnp.bfloat16),
        ],
        compiler_params=pltpu.CompilerParams(
            dimension_semantics=("parallel",)),
        cost_estimate=pl.CostEstimate(flops=flops, transcendentals=0,
                                      bytes_accessed=nbytes),
    )(*ins)

    out = out_par.reshape(g, 2, 4, half, hw).transpose(0, 1, 3, 2, 4)
    out = out.reshape(n, 4, hw)
    out = out.reshape(n, 2, 2, h, w).transpose(0, 3, 1, 4, 2)
    return out.reshape(n, 1, 2 * h, 2 * w)


# speedup vs baseline: 2.0862x; 1.1611x over previous
import jax
import jax.numpy as jnp
from jax.experimental import pallas as pl
from jax.experimental.pallas import tpu as pltpu

_DEPTH = 7
_MID = 4
_C8 = 2 * _MID
_KB = 9 * 2 * _C8
_KU = _DEPTH * _KB
_SLOT = 16
_UP_ROW = _SLOT * (1 + _DEPTH)
_BIL_ROW = _UP_ROW + _SLOT
_SLAB_ROWS = _BIL_ROW + _SLOT
_SLAB_COLS = 1024
_XTC = 32


def _cnn_kernel_body(h, w, n2):
    taps = [(t // 3 - 1, t % 3 - 1) for t in range(9)]

    def body(x_ref, w_ref, b_ref, m_ref, out_ref,
             xt_ref, ct_ref, ft_ref):

        def write_taps(act_bf, dst_ref, base):
            for t, (di, dj) in enumerate(taps):
                s = di * w + dj
                if s == 0:
                    dst_ref[base + 16 * t: base + 16 * (t + 1), :] = act_bf
                else:
                    v = pltpu.roll(act_bf, (-s) % n2, axis=1)
                    dst_ref[base + 16 * t: base + 16 * (t + 1), :] = (
                        v * m_ref[16 * t:16 * (t + 1), :])

        x_bf = x_ref[0].astype(jnp.bfloat16)
        for t, (di, dj) in enumerate(taps):
            s = di * w + dj
            if s == 0:
                xt_ref[2 * t:2 * t + 2, :] = x_bf
            else:
                v = pltpu.roll(x_bf, (-s) % n2, axis=1)
                xt_ref[2 * t:2 * t + 2, :] = v * m_ref[16 * t:16 * t + 2, :]

        xcol = {}
        for dj in (-1, 0, 1):
            if dj == 0:
                xcol[0] = x_bf
            else:
                r = pltpu.roll(x_bf, (-dj) % n2, axis=1)
                mc = m_ref[16 * (4 + dj):16 * (4 + dj) + 2, :]
                xcol[dj] = x_bf + mc * (r - x_bf)
        for t, (di, dj) in enumerate(taps):
            v = xcol[dj]
            if di != 0:
                r = pltpu.roll(v, (-di * w) % n2, axis=1)
                mr = m_ref[16 * (3 * di + 4):16 * (3 * di + 4) + 2, :]
                v = v + mr * (r - v)
            xt_ref[_XTC + 2 * t:_XTC + 2 * t + 2, :] = v

        h0 = jnp.dot(w_ref[0:_SLOT, 0:18], xt_ref[0:18, :],
                     preferred_element_type=jnp.float32)
        h0 = h0 + b_ref[0:_SLOT, :]
        write_taps(h0.astype(jnp.bfloat16), ct_ref, 0)

        for i in range(_DEPTH):
            rhs = ct_ref[...] if i == 0 else ft_ref[_KB * (i - 1):_KB * i, :]
            r = _SLOT * (1 + i)
            y = jnp.dot(w_ref[r:r + _SLOT, 0:_KB], rhs,
                        preferred_element_type=jnp.float32)
            crelu = jnp.maximum(y + b_ref[r:r + _SLOT, :], 0.0)
            write_taps(crelu.astype(jnp.bfloat16), ft_ref, _KB * i)

        up = jnp.dot(w_ref[_UP_ROW:_UP_ROW + 8, 0:_KU], ft_ref[...],
                     preferred_element_type=jnp.float32)
        up = up + b_ref[_UP_ROW:_UP_ROW + 8, :]

        res = jnp.dot(w_ref[_BIL_ROW:_BIL_ROW + 8, 0:18],
                      xt_ref[_XTC:_XTC + 18, :],
                      preferred_element_type=jnp.float32)
        out_ref[0] = (up + res).astype(out_ref.dtype)

    return body


def _fold_upscaler(w_up):
    m = jnp.array([[[1., 0., 0.], [0., 1., 1.], [0., 0., 0.]],
                   [[0., 0., 0.], [1., 1., 0.], [0., 0., 1.]]],
                  dtype=w_up.dtype)
    k = jnp.einsum("pia,qjb,ocab->pqocij", m, m, w_up)
    k = jnp.transpose(k[:, :, 0], (3, 4, 0, 1, 2))
    return k.reshape(9, 4, w_up.shape[1])


def _pack_slabs(params):
    wslab = jnp.zeros((_SLAB_ROWS, _SLAB_COLS), jnp.float32)
    bslab = jnp.zeros((_SLAB_ROWS, 1), jnp.float32)

    w0 = params["conv_layer"]["w"].reshape(_C8, 9)
    s0 = jnp.zeros((_SLOT, 9, 2), jnp.float32)
    for p in range(2):
        s0 = s0.at[_C8 * p:_C8 * (p + 1), :, p].set(w0)
    wslab = wslab.at[0:_SLOT, 0:18].set(s0.reshape(_SLOT, 18))
    bslab = bslab.at[0:_SLOT, 0].set(
        jnp.tile(params["conv_layer"]["b"], 2))

    for i, blk in enumerate(params["blocks"]):
        wb = jnp.transpose(blk["w"], (0, 2, 3, 1)).reshape(_MID, 9, _C8)
        sb = jnp.zeros((_SLOT, 9, 2, _C8), jnp.float32)
        for p in range(2):
            sb = sb.at[_C8 * p:_C8 * p + _MID, :, p, :].set(wb)
            sb = sb.at[_C8 * p + _MID:_C8 * (p + 1), :, p, :].set(-wb)
        r = _SLOT * (1 + i)
        wslab = wslab.at[r:r + _SLOT, 0:_KB].set(sb.reshape(_SLOT, _KB))
        bb = jnp.concatenate([blk["b"], -blk["b"]])
        bslab = bslab.at[r:r + _SLOT, 0].set(jnp.tile(bb, 2))

    wu = _fold_upscaler(params["upscaler"]["w"])
    wu = jnp.transpose(wu.reshape(9, 4, _DEPTH, _C8), (1, 2, 0, 3))
    su = jnp.zeros((8, _DEPTH, 9, 2, _C8), jnp.float32)
    for p in range(2):
        su = su.at[4 * p:4 * (p + 1), :, :, p, :].set(wu)
    wslab = wslab.at[_UP_ROW:_UP_ROW + 8, 0:_KU].set(su.reshape(8, _KU))
    bslab = bslab.at[_UP_ROW:_UP_ROW + 8, 0].set(params["upscaler"]["b"][0])

    wr = jnp.array([[0.25, 0.75, 0.0], [0.0, 0.75, 0.25]], jnp.float32)
    wb = jnp.einsum("pi,qj->pqij", wr, wr).reshape(4, 9)
    sr = jnp.zeros((8, 9, 2), jnp.float32)
    for p in range(2):
        sr = sr.at[4 * p:4 * (p + 1), :, p].set(wb)
    wslab = wslab.at[_BIL_ROW:_BIL_ROW + 8, 0:18].set(sr.reshape(8, 18))
    return wslab.astype(jnp.bfloat16), bslab


def _mask_slab(h, w, imgs):
    hw = h * w
    q = jnp.arange(hw, dtype=jnp.int32)
    yy, xx = q // w, q % w
    rows = []
    for t in range(9):
        di, dj = t // 3 - 1, t % 3 - 1
        ok = (yy + di >= 0) & (yy + di < h) & (xx + dj >= 0) & (xx + dj < w)
        rows.append(ok)
    m = jnp.stack(rows).astype(jnp.bfloat16)
    m = jnp.repeat(m, _SLOT, axis=0)
    return jnp.tile(m, (1, imgs))


def _pick_bsub(n):
    for b in (16, 8, 4, 2):
        if n % b == 0 and n // b >= 2:
            return b
    return 1


def kernel(x, conv_layer_w, conv_layer_b, block0_w, block0_b, block1_w,
           block1_b, block2_w, block2_b, block3_w, block3_b, block4_w,
           block4_b, block5_w, block5_b, block6_w, block6_b, upscaler_w,
           upscaler_b):
    params = {
        "conv_layer": {"w": conv_layer_w, "b": conv_layer_b},
        "blocks": [
            {"w": block0_w, "b": block0_b}, {"w": block1_w, "b": block1_b},
            {"w": block2_w, "b": block2_b}, {"w": block3_w, "b": block3_b},
            {"w": block4_w, "b": block4_b}, {"w": block5_w, "b": block5_b},
            {"w": block6_w, "b": block6_b},
        ],
        "upscaler": {"w": upscaler_w, "b": upscaler_b},
    }
    n, c, h, w = x.shape
    hw = h * w
    bsub = _pick_bsub(n)
    if bsub == 1:
        raise ValueError("batch must be even")
    g = n // bsub
    half = bsub // 2
    n2 = half * hw

    wslab, bslab = _pack_slabs(params)
    masks = _mask_slab(h, w, half)
    x_g = x.reshape(g, 2, n2)

    flops = 2 * n * hw * (_C8 * 9 + _DEPTH * _C8 * _KB // 2 + 2 * _KU)
    ins = (x_g, wslab, bslab, masks)
    nbytes = sum(a.size * a.dtype.itemsize for a in ins) + n * 4 * hw * 4

    out_par = pl.pallas_call(
        _cnn_kernel_body(h, w, n2),
        out_shape=jax.ShapeDtypeStruct((g, 8, n2), x.dtype),
        grid=(g,),
        in_specs=[
            pl.BlockSpec((1, 2, n2), lambda i: (i, 0, 0)),
            pl.BlockSpec((_SLAB_ROWS, _SLAB_COLS), lambda i: (0, 0)),
            pl.BlockSpec((_SLAB_ROWS, 1), lambda i: (0, 0)),
            pl.BlockSpec((_SLOT * 9, n2), lambda i: (0, 0)),
        ],
        out_specs=pl.BlockSpec((1, 8, n2), lambda i: (i, 0, 0)),
        scratch_shapes=[
            pltpu.VMEM((64, n2), jnp.bfloat16),
            pltpu.VMEM((_KB, n2), jnp.bfloat16),
            pltpu.VMEM((_KU, n2), jnp.bfloat16),
        ],
        compiler_params=pltpu.CompilerParams(
            dimension_semantics=("parallel",)),
        cost_estimate=pl.CostEstimate(flops=flops, transcendentals=0,
                                      bytes_accessed=nbytes),
    )(*ins)

    out = out_par.reshape(g, 2, 4, half, hw).transpose(0, 1, 3, 2, 4)
    out = out.reshape(n, 4, hw)
    out = out.reshape(n, 2, 2, h, w).transpose(0, 3, 1, 4, 2)
    return out.reshape(n, 1, 2 * h, 2 * w)


# speedup vs baseline: 2.6949x; 1.2918x over previous
import numpy as np

import jax
import jax.numpy as jnp
from jax.experimental import pallas as pl
from jax.experimental.pallas import tpu as pltpu
from jax.sharding import Mesh, PartitionSpec as P

_DEPTH = 7
_MID = 4
_C8 = 2 * _MID
_KB = 9 * 2 * _C8
_KU = _DEPTH * _KB
_SLOT = 16
_UP_ROW = _SLOT * (1 + _DEPTH)
_BIL_ROW = _UP_ROW + _SLOT
_SLAB_ROWS = _BIL_ROW + _SLOT
_SLAB_COLS = 1024
_XTC = 32


def _cnn_kernel_body(h, w, n2):
    taps = [(t // 3 - 1, t % 3 - 1) for t in range(9)]

    def body(x_ref, w_ref, b_ref, m_ref, out_ref,
             xt_ref, ct_ref, ft_ref):

        def write_taps(act_bf, dst_ref, base):
            for t, (di, dj) in enumerate(taps):
                s = di * w + dj
                if s == 0:
                    dst_ref[base + 16 * t: base + 16 * (t + 1), :] = act_bf
                else:
                    v = pltpu.roll(act_bf, (-s) % n2, axis=1)
                    dst_ref[base + 16 * t: base + 16 * (t + 1), :] = (
                        v * m_ref[16 * t:16 * (t + 1), :])

        x_bf = x_ref[0].astype(jnp.bfloat16)
        for t, (di, dj) in enumerate(taps):
            s = di * w + dj
            if s == 0:
                xt_ref[2 * t:2 * t + 2, :] = x_bf
            else:
                v = pltpu.roll(x_bf, (-s) % n2, axis=1)
                xt_ref[2 * t:2 * t + 2, :] = v * m_ref[16 * t:16 * t + 2, :]

        xcol = {}
        for dj in (-1, 0, 1):
            if dj == 0:
                xcol[0] = x_bf
            else:
                r = pltpu.roll(x_bf, (-dj) % n2, axis=1)
                mc = m_ref[16 * (4 + dj):16 * (4 + dj) + 2, :]
                xcol[dj] = x_bf + mc * (r - x_bf)
        for t, (di, dj) in enumerate(taps):
            v = xcol[dj]
            if di != 0:
                r = pltpu.roll(v, (-di * w) % n2, axis=1)
                mr = m_ref[16 * (3 * di + 4):16 * (3 * di + 4) + 2, :]
                v = v + mr * (r - v)
            xt_ref[_XTC + 2 * t:_XTC + 2 * t + 2, :] = v

        h0 = jnp.dot(w_ref[0:_SLOT, 0:18], xt_ref[0:18, :],
                     preferred_element_type=jnp.float32)
        h0 = h0 + b_ref[0:_SLOT, :]
        write_taps(h0.astype(jnp.bfloat16), ct_ref, 0)

        for i in range(_DEPTH):
            rhs = ct_ref[...] if i == 0 else ft_ref[_KB * (i - 1):_KB * i, :]
            r = _SLOT * (1 + i)
            y = jnp.dot(w_ref[r:r + _SLOT, 0:_KB], rhs,
                        preferred_element_type=jnp.float32)
            crelu = jnp.maximum(y + b_ref[r:r + _SLOT, :], 0.0)
            write_taps(crelu.astype(jnp.bfloat16), ft_ref, _KB * i)

        up = jnp.dot(w_ref[_UP_ROW:_UP_ROW + 8, 0:_KU], ft_ref[...],
                     preferred_element_type=jnp.float32)
        up = up + b_ref[_UP_ROW:_UP_ROW + 8, :]

        res = jnp.dot(w_ref[_BIL_ROW:_BIL_ROW + 8, 0:18],
                      xt_ref[_XTC:_XTC + 18, :],
                      preferred_element_type=jnp.float32)
        out_ref[0] = (up + res).astype(out_ref.dtype)

    return body


def _fold_upscaler(w_up):
    m = jnp.array([[[1., 0., 0.], [0., 1., 1.], [0., 0., 0.]],
                   [[0., 0., 0.], [1., 1., 0.], [0., 0., 1.]]],
                  dtype=w_up.dtype)
    k = jnp.einsum("pia,qjb,ocab->pqocij", m, m, w_up)
    k = jnp.transpose(k[:, :, 0], (3, 4, 0, 1, 2))
    return k.reshape(9, 4, w_up.shape[1])


def _pack_slabs(params):
    wslab = jnp.zeros((_SLAB_ROWS, _SLAB_COLS), jnp.float32)
    bslab = jnp.zeros((_SLAB_ROWS, 1), jnp.float32)

    w0 = params["conv_layer"]["w"].reshape(_C8, 9)
    s0 = jnp.zeros((_SLOT, 9, 2), jnp.float32)
    for p in range(2):
        s0 = s0.at[_C8 * p:_C8 * (p + 1), :, p].set(w0)
    wslab = wslab.at[0:_SLOT, 0:18].set(s0.reshape(_SLOT, 18))
    bslab = bslab.at[0:_SLOT, 0].set(
        jnp.tile(params["conv_layer"]["b"], 2))

    for i, blk in enumerate(params["blocks"]):
        wb = jnp.transpose(blk["w"], (0, 2, 3, 1)).reshape(_MID, 9, _C8)
        sb = jnp.zeros((_SLOT, 9, 2, _C8), jnp.float32)
        for p in range(2):
            sb = sb.at[_C8 * p:_C8 * p + _MID, :, p, :].set(wb)
            sb = sb.at[_C8 * p + _MID:_C8 * (p + 1), :, p, :].set(-wb)
        r = _SLOT * (1 + i)
        wslab = wslab.at[r:r + _SLOT, 0:_KB].set(sb.reshape(_SLOT, _KB))
        bb = jnp.concatenate([blk["b"], -blk["b"]])
        bslab = bslab.at[r:r + _SLOT, 0].set(jnp.tile(bb, 2))

    wu = _fold_upscaler(params["upscaler"]["w"])
    wu = jnp.transpose(wu.reshape(9, 4, _DEPTH, _C8), (1, 2, 0, 3))
    su = jnp.zeros((8, _DEPTH, 9, 2, _C8), jnp.float32)
    for p in range(2):
        su = su.at[4 * p:4 * (p + 1), :, :, p, :].set(wu)
    wslab = wslab.at[_UP_ROW:_UP_ROW + 8, 0:_KU].set(su.reshape(8, _KU))
    bslab = bslab.at[_UP_ROW:_UP_ROW + 8, 0].set(params["upscaler"]["b"][0])

    wr = jnp.array([[0.25, 0.75, 0.0], [0.0, 0.75, 0.25]], jnp.float32)
    wb = jnp.einsum("pi,qj->pqij", wr, wr).reshape(4, 9)
    sr = jnp.zeros((8, 9, 2), jnp.float32)
    for p in range(2):
        sr = sr.at[4 * p:4 * (p + 1), :, p].set(wb)
    wslab = wslab.at[_BIL_ROW:_BIL_ROW + 8, 0:18].set(sr.reshape(8, 18))
    return wslab.astype(jnp.bfloat16), bslab


def _mask_slab(h, w, imgs):
    hw = h * w
    q = jnp.arange(hw, dtype=jnp.int32)
    yy, xx = q // w, q % w
    rows = []
    for t in range(9):
        di, dj = t // 3 - 1, t % 3 - 1
        ok = (yy + di >= 0) & (yy + di < h) & (xx + dj >= 0) & (xx + dj < w)
        rows.append(ok)
    m = jnp.stack(rows).astype(jnp.bfloat16)
    m = jnp.repeat(m, _SLOT, axis=0)
    return jnp.tile(m, (1, imgs))


def _pick_bsub(n):
    for b in (16, 8, 4, 2):
        if n % b == 0 and n // b >= 2:
            return b
    return 1


def kernel(x, conv_layer_w, conv_layer_b, block0_w, block0_b, block1_w,
           block1_b, block2_w, block2_b, block3_w, block3_b, block4_w,
           block4_b, block5_w, block5_b, block6_w, block6_b, upscaler_w,
           upscaler_b):
    params = {
        "conv_layer": {"w": conv_layer_w, "b": conv_layer_b},
        "blocks": [
            {"w": block0_w, "b": block0_b}, {"w": block1_w, "b": block1_b},
            {"w": block2_w, "b": block2_b}, {"w": block3_w, "b": block3_b},
            {"w": block4_w, "b": block4_b}, {"w": block5_w, "b": block5_b},
            {"w": block6_w, "b": block6_b},
        ],
        "upscaler": {"w": upscaler_w, "b": upscaler_b},
    }
    n, c, h, w = x.shape
    hw = h * w
    bsub = _pick_bsub(n)
    if bsub == 1:
        raise ValueError("batch must be even")
    g = n // bsub
    half = bsub // 2
    n2 = half * hw

    wslab, bslab = _pack_slabs(params)
    masks = _mask_slab(h, w, half)
    x_g = x.reshape(g, 2, n2)

    flops = 2 * n * hw * (_C8 * 9 + _DEPTH * _C8 * _KB // 2 + 2 * _KU)
    nbytes = (x.size * 4 + wslab.size * 2 + bslab.size * 4 + masks.size * 2
              + n * 4 * hw * 4)

    def run(x_s, w_s, b_s, m_s):
        gs = x_s.shape[0]
        return pl.pallas_call(
            _cnn_kernel_body(h, w, n2),
            out_shape=jax.ShapeDtypeStruct((gs, 8, n2), x.dtype),
            grid=(gs,),
            in_specs=[
                pl.BlockSpec((1, 2, n2), lambda i: (i, 0, 0)),
                pl.BlockSpec((_SLAB_ROWS, _SLAB_COLS), lambda i: (0, 0)),
                pl.BlockSpec((_SLAB_ROWS, 1), lambda i: (0, 0)),
                pl.BlockSpec((_SLOT * 9, n2), lambda i: (0, 0)),
            ],
            out_specs=pl.BlockSpec((1, 8, n2), lambda i: (i, 0, 0)),
            scratch_shapes=[
                pltpu.VMEM((64, n2), jnp.bfloat16),
                pltpu.VMEM((_KB, n2), jnp.bfloat16),
                pltpu.VMEM((_KU, n2), jnp.bfloat16),
            ],
            compiler_params=pltpu.CompilerParams(
                dimension_semantics=("parallel",)),
            cost_estimate=pl.CostEstimate(
                flops=flops // (g // gs), transcendentals=0,
                bytes_accessed=nbytes // (g // gs)),
        )(x_s, w_s, b_s, m_s)

    ndev = jax.device_count()
    if ndev > 1 and g % ndev == 0:
        mesh = Mesh(np.asarray(jax.devices()), ("d",))
        run = jax.shard_map(run, mesh=mesh,
                            in_specs=(P("d"), P(), P(), P()),
                            out_specs=P("d"), check_vma=False)
    out_par = run(x_g, wslab, bslab, masks)

    out = out_par.reshape(g, 2, 4, half, hw).transpose(0, 1, 3, 2, 4)
    out = out.reshape(n, 4, hw)
    out = out.reshape(n, 2, 2, h, w).transpose(0, 3, 1, 4, 2)
    return out.reshape(n, 1, 2 * h, 2 * w)


# speedup vs baseline: 2.9257x; 1.0856x over previous
import numpy as np

import jax
import jax.numpy as jnp
from jax.experimental import pallas as pl
from jax.experimental.pallas import tpu as pltpu
from jax.sharding import Mesh, PartitionSpec as P

_DEPTH = 7
_MID = 4
_C8 = 2 * _MID
_KB = 9 * 2 * _C8
_KU = _DEPTH * _KB
_SLOT = 16
_UP_ROW = _SLOT * (1 + _DEPTH)
_BIL_ROW = _UP_ROW + _SLOT
_SLAB_ROWS = _BIL_ROW + _SLOT
_SLAB_COLS = 1024
_XTC = 32


def _cnn_kernel_body(h, w, n2):
    taps = [(t // 3 - 1, t % 3 - 1) for t in range(9)]

    def body(x_ref, w_ref, b_ref, m_ref, out_ref,
             xt_ref, ct_ref, ft_ref):

        def write_taps(act_bf, dst_ref, base):
            for t, (di, dj) in enumerate(taps):
                s = di * w + dj
                if s == 0:
                    dst_ref[base + 16 * t: base + 16 * (t + 1), :] = act_bf
                else:
                    v = pltpu.roll(act_bf, (-s) % n2, axis=1)
                    dst_ref[base + 16 * t: base + 16 * (t + 1), :] = (
                        v * m_ref[16 * t:16 * (t + 1), :])

        x_bf = x_ref[0].astype(jnp.bfloat16)
        for t, (di, dj) in enumerate(taps):
            s = di * w + dj
            if s == 0:
                xt_ref[2 * t:2 * t + 2, :] = x_bf
            else:
                v = pltpu.roll(x_bf, (-s) % n2, axis=1)
                xt_ref[2 * t:2 * t + 2, :] = v * m_ref[16 * t:16 * t + 2, :]

        xcol = {}
        for dj in (-1, 0, 1):
            if dj == 0:
                xcol[0] = x_bf
            else:
                r = pltpu.roll(x_bf, (-dj) % n2, axis=1)
                mc = m_ref[16 * (4 + dj):16 * (4 + dj) + 2, :]
                xcol[dj] = x_bf + mc * (r - x_bf)
        for t, (di, dj) in enumerate(taps):
            v = xcol[dj]
            if di != 0:
                r = pltpu.roll(v, (-di * w) % n2, axis=1)
                mr = m_ref[16 * (3 * di + 4):16 * (3 * di + 4) + 2, :]
                v = v + mr * (r - v)
            xt_ref[_XTC + 2 * t:_XTC + 2 * t + 2, :] = v

        h0 = jnp.dot(w_ref[0:_SLOT, 0:18], xt_ref[0:18, :],
                     preferred_element_type=jnp.float32)
        h0 = h0 + b_ref[0:_SLOT, :]
        write_taps(h0.astype(jnp.bfloat16), ct_ref, 0)

        for i in range(_DEPTH):
            rhs = ct_ref[...] if i == 0 else ft_ref[_KB * (i - 1):_KB * i, :]
            r = _SLOT * (1 + i)
            y = jnp.dot(w_ref[r:r + _SLOT, 0:_KB], rhs,
                        preferred_element_type=jnp.float32)
            crelu = jnp.maximum(y + b_ref[r:r + _SLOT, :], 0.0)
            write_taps(crelu.astype(jnp.bfloat16), ft_ref, _KB * i)

        up = jnp.dot(w_ref[_UP_ROW:_UP_ROW + 8, 0:_KU], ft_ref[...],
                     preferred_element_type=jnp.float32)
        up = up + b_ref[_UP_ROW:_UP_ROW + 8, :]

        res = jnp.dot(w_ref[_BIL_ROW:_BIL_ROW + 8, 0:18],
                      xt_ref[_XTC:_XTC + 18, :],
                      preferred_element_type=jnp.float32)
        out_ref[0] = (up + res).astype(out_ref.dtype)

    return body


def _fold_upscaler(w_up):
    m = jnp.array([[[1., 0., 0.], [0., 1., 1.], [0., 0., 0.]],
                   [[0., 0., 0.], [1., 1., 0.], [0., 0., 1.]]],
                  dtype=w_up.dtype)
    k = jnp.einsum("pia,qjb,ocab->pqocij", m, m, w_up)
    k = jnp.transpose(k[:, :, 0], (3, 4, 0, 1, 2))
    return k.reshape(9, 4, w_up.shape[1])


def _pack_slabs(params):
    eye = np.eye(2, dtype=np.float32)

    w0 = params["conv_layer"]["w"].reshape(_C8, 9)
    s0 = jnp.einsum("ot,pq->potq", w0, eye).reshape(_SLOT, 18)

    wb7 = jnp.stack([b["w"] for b in params["blocks"]])
    wb7 = jnp.transpose(wb7, (0, 1, 3, 4, 2)).reshape(_DEPTH, _MID, 9, _C8)
    wb7 = jnp.concatenate([wb7, -wb7], axis=1)
    sb = jnp.einsum("iotc,pq->ipotqc", wb7, eye).reshape(
        _DEPTH * _SLOT, _KB)

    wu = _fold_upscaler(params["upscaler"]["w"])
    wu = jnp.transpose(wu.reshape(9, 4, _DEPTH, _C8), (1, 2, 0, 3))
    su = jnp.einsum("fitc,pq->pfitqc", wu, eye).reshape(8, _KU)

    wr = np.array([[0.25, 0.75, 0.0], [0.0, 0.75, 0.25]], np.float32)
    wbil = np.einsum("pi,qj->pqij", wr, wr).reshape(4, 9)
    sr = jnp.einsum("ft,pq->pftq", jnp.asarray(wbil), eye).reshape(8, 18)

    def slot(a, rows):
        a = a.astype(jnp.bfloat16)
        return jnp.pad(a, ((0, rows - a.shape[0]),
                           (0, _SLAB_COLS - a.shape[1])))

    wslab = jnp.concatenate(
        [slot(s0, _SLOT), slot(sb, _DEPTH * _SLOT),
         slot(su, _SLOT), slot(sr, _SLOT)], axis=0)

    bb7 = jnp.concatenate(
        [jnp.stack([b["b"] for b in params["blocks"]]),
         -jnp.stack([b["b"] for b in params["blocks"]])], axis=1)
    bias = jnp.concatenate([
        jnp.tile(params["conv_layer"]["b"], 2),
        jnp.tile(bb7, (1, 2)).reshape(_DEPTH * _SLOT),
        jnp.broadcast_to(params["upscaler"]["b"], (_SLOT,)),
    ])
    return wslab, bias.astype(jnp.float32)[:, None]


def _mask_slab(h, w, imgs):
    hw = h * w
    q = np.arange(hw, dtype=np.int32)
    yy, xx = q // w, q % w
    rows = []
    for t in range(9):
        di, dj = t // 3 - 1, t % 3 - 1
        ok = (yy + di >= 0) & (yy + di < h) & (xx + dj >= 0) & (xx + dj < w)
        rows.append(ok)
    m = np.stack(rows).astype(np.float32)
    m = np.repeat(m, _SLOT, axis=0)
    return jnp.asarray(np.tile(m, (1, imgs)), dtype=jnp.bfloat16)


def _pick_bsub(n):
    for b in (16, 8, 4, 2):
        if n % b == 0 and n // b >= 2:
            return b
    return 1


def kernel(x, conv_layer_w, conv_layer_b, block0_w, block0_b, block1_w,
           block1_b, block2_w, block2_b, block3_w, block3_b, block4_w,
           block4_b, block5_w, block5_b, block6_w, block6_b, upscaler_w,
           upscaler_b):
    params = {
        "conv_layer": {"w": conv_layer_w, "b": conv_layer_b},
        "blocks": [
            {"w": block0_w, "b": block0_b}, {"w": block1_w, "b": block1_b},
            {"w": block2_w, "b": block2_b}, {"w": block3_w, "b": block3_b},
            {"w": block4_w, "b": block4_b}, {"w": block5_w, "b": block5_b},
            {"w": block6_w, "b": block6_b},
        ],
        "upscaler": {"w": upscaler_w, "b": upscaler_b},
    }
    n, c, h, w = x.shape
    hw = h * w
    bsub = _pick_bsub(n)
    if bsub == 1:
        raise ValueError("batch must be even")
    g = n // bsub
    half = bsub // 2
    n2 = half * hw

    wslab, bslab = _pack_slabs(params)
    masks = _mask_slab(h, w, half)
    x_g = x.reshape(g, 2, n2)

    flops = 2 * n * hw * (_C8 * 9 + _DEPTH * _C8 * _KB // 2 + 2 * _KU)
    nbytes = (x.size * 4 + wslab.size * 2 + bslab.size * 4 + masks.size * 2
              + n * 4 * hw * 4)

    def run(x_s, w_s, b_s, m_s):
        gs = x_s.shape[0]
        return pl.pallas_call(
            _cnn_kernel_body(h, w, n2),
            out_shape=jax.ShapeDtypeStruct((gs, 8, n2), x.dtype),
            grid=(gs,),
            in_specs=[
                pl.BlockSpec((1, 2, n2), lambda i: (i, 0, 0)),
                pl.BlockSpec((_SLAB_ROWS, _SLAB_COLS), lambda i: (0, 0)),
                pl.BlockSpec((_SLOT * 9, 1), lambda i: (0, 0)),
                pl.BlockSpec((_SLOT * 9, n2), lambda i: (0, 0)),
            ],
            out_specs=pl.BlockSpec((1, 8, n2), lambda i: (i, 0, 0)),
            scratch_shapes=[
                pltpu.VMEM((64, n2), jnp.bfloat16),
                pltpu.VMEM((_KB, n2), jnp.bfloat16),
                pltpu.VMEM((_KU, n2), jnp.bfloat16),
            ],
            compiler_params=pltpu.CompilerParams(
                dimension_semantics=("parallel",)),
            cost_estimate=pl.CostEstimate(
                flops=flops // (g // gs), transcendentals=0,
                bytes_accessed=nbytes // (g // gs)),
        )(x_s, w_s, b_s, m_s)

    ndev = jax.device_count()
    if ndev > 1 and g % ndev == 0:
        mesh = Mesh(np.asarray(jax.devices()), ("d",))
        run = jax.shard_map(run, mesh=mesh,
                            in_specs=(P("d"), P(), P(), P()),
                            out_specs=P("d"), check_vma=False)
    out_par = run(x_g, wslab, bslab, masks)

    out = out_par.reshape(g, 2, 4, half, hw).transpose(0, 1, 3, 2, 4)
    out = out.reshape(n, 4, hw)
    out = out.reshape(n, 2, 2, h, w).transpose(0, 3, 1, 4, 2)
    return out.reshape(n, 1, 2 * h, 2 * w)
